# baseline (speedup 1.0000x reference)
import functools

import jax
import jax.numpy as jnp
from jax import lax
from jax.experimental import pallas as pl
from jax.experimental.pallas import tpu as pltpu
from jax.experimental.pallas import tpu_sc as plsc

N = 10000
NPAD = 10240
D = 128
NC = 2
NS = 16
NW = NC * NS
B = 128
TOTB = 2560
EPAD = TOTB * B
RPS = NPAD // NS
CH = 8
NCH_TOT = TOTB // CH
ZROWS = 32
Q0 = 20
Q1 = NCH_TOT // NS - Q0

_sc_mesh = plsc.VectorSubcoreMesh(core_axis_name="c", subcore_axis_name="s")


HIST_B = TOTB // NW


@functools.partial(
    pl.kernel,
    out_type=jax.ShapeDtypeStruct((NC, NPAD), jnp.float32),
    mesh=_sc_mesh,
    scratch_types=[
        pltpu.VMEM((HIST_B, B), jnp.int32),
        pltpu.VMEM((B,), jnp.float32),
        pltpu.VMEM((RPS,), jnp.float32),
        pltpu.VMEM_SHARED((NPAD,), jnp.float32),
    ],
)
def _sc_hist(dst_hbm, out_hbm, dst_v, ones_v, zb_v, hist_sh):
    c = lax.axis_index("c")
    s = lax.axis_index("s")
    wid = s * NC + c

    def fill_ones(i, carry):
        ones_v[pl.ds(i * 16, 16)] = jnp.full((16,), 1.0, jnp.float32)
        return carry

    lax.fori_loop(0, B // 16, fill_ones, 0)

    def fill_zeros(i, carry):
        zb_v[pl.ds(i * 16, 16)] = jnp.zeros((16,), jnp.float32)
        return carry

    lax.fori_loop(0, RPS // 16, fill_zeros, 0)

    pltpu.sync_copy(zb_v, hist_sh.at[pl.ds(s * RPS, RPS)])
    plsc.subcore_barrier()

    pltpu.sync_copy(dst_hbm.at[pl.ds(wid * HIST_B, HIST_B)], dst_v)

    def body(j, carry):
        pltpu.sync_copy(ones_v, hist_sh.at[dst_v.at[j]], add=True)
        return carry

    lax.fori_loop(0, HIST_B, body, 0)
    plsc.subcore_barrier()
    pltpu.sync_copy(hist_sh.at[pl.ds(s * RPS, RPS)],
                    out_hbm.at[c, pl.ds(s * RPS, RPS)])


@functools.partial(
    pl.kernel,
    out_type=jax.ShapeDtypeStruct((NC, NPAD, D), jnp.float32),
    mesh=_sc_mesh,
    scratch_types=[
        pltpu.VMEM((2, CH, B), jnp.int32),
        pltpu.VMEM((2, CH, B), jnp.int32),
        pltpu.VMEM((2, B, D), jnp.float32),
        pltpu.VMEM((ZROWS, D), jnp.float32),
        pltpu.VMEM_SHARED((NPAD, D), jnp.float32),
        pltpu.SemaphoreType.DMA,
        pltpu.SemaphoreType.DMA,
        pltpu.SemaphoreType.DMA,
        pltpu.SemaphoreType.DMA,
    ],
)
def _sc_agg(hs_hbm, src_hbm, dst_hbm, out_hbm,
            src_v, dst_v, gbuf, zb, acc_sh, sem_g0, sem_g1, sem_i0, sem_i1):
    c = lax.axis_index("c")
    s = lax.axis_index("s")
    sem_g = (sem_g0, sem_g1)
    sem_i = (sem_i0, sem_i1)
    nch = jnp.where(c == 0, Q0, Q1)
    cbase = jnp.where(c == 0, s * Q0, NS * Q0 + s * Q1)

    def issue_idx(ib, chunk):
        pltpu.async_copy(src_hbm.at[pl.ds(chunk * CH, CH)],
                         src_v.at[ib], sem_i[ib])
        pltpu.async_copy(dst_hbm.at[pl.ds(chunk * CH, CH)],
                         dst_v.at[ib], sem_i[ib])

    def wait_idx(ib):
        pltpu.make_async_copy(src_hbm.at[pl.ds(0, CH)],
                              src_v.at[ib], sem_i[ib]).wait()
        pltpu.make_async_copy(dst_hbm.at[pl.ds(0, CH)],
                              dst_v.at[ib], sem_i[ib]).wait()

    def issue_gather(k):
        pltpu.async_copy(hs_hbm.at[src_v.at[k // CH, k % CH]],
                         gbuf.at[k % 2], sem_g[k % 2])

    def wait_gather(k):
        pltpu.make_async_copy(hs_hbm.at[src_v.at[k // CH, k % CH]],
                              gbuf.at[k % 2], sem_g[k % 2]).wait()

    @pl.when(nch > 0)
    def _():
        issue_idx(0, cbase)
        issue_idx(1, cbase + 1)

    def fill_zeros(r, carry):
        for k in range(D // 16):
            zb[r, pl.ds(k * 16, 16)] = jnp.zeros((16,), jnp.float32)
        return carry

    lax.fori_loop(0, ZROWS, fill_zeros, 0)

    def zero_acc(t, carry):
        pltpu.sync_copy(zb, acc_sh.at[pl.ds(s * RPS + t * ZROWS, ZROWS)])
        return carry

    lax.fori_loop(0, RPS // ZROWS, zero_acc, 0)
    plsc.subcore_barrier()

    def body(u, carry):
        wait_idx(0)
        issue_gather(0)
        for k in range(2 * CH):
            if k + 1 < 2 * CH:
                if k + 1 == CH:
                    wait_idx(1)
                issue_gather(k + 1)
            wait_gather(k)
            pltpu.sync_copy(gbuf.at[k % 2],
                            acc_sh.at[dst_v.at[k // CH, k % CH]], add=True)
            if k == CH - 1:
                @pl.when(u < nch // 2 - 1)
                def _():
                    issue_idx(0, cbase + 2 * u + 2)

        @pl.when(u < nch // 2 - 1)
        def _():
            issue_idx(1, cbase + 2 * u + 3)
        return carry

    lax.fori_loop(0, nch // 2, body, 0)

    plsc.subcore_barrier()
    pltpu.sync_copy(acc_sh.at[pl.ds(s * RPS, RPS)],
                    out_hbm.at[c, pl.ds(s * RPS, RPS)])


BR = 1024
_TC_GRID = NPAD // BR


def _tc_dense1_body(x_ref, w_ref, h0_ref, h1_ref, hs_ref, dis_ref):
    deg = h0_ref[...] + h1_ref[...] + 1.0
    dis = lax.rsqrt(deg)
    rows = pl.program_id(0) * BR + lax.broadcasted_iota(jnp.int32, (BR,), 0)
    dis = jnp.where(rows < N, dis, 0.0)
    h = jnp.dot(x_ref[...], w_ref[...], preferred_element_type=jnp.float32)
    hs_ref[...] = h * dis[:, None]
    dis_ref[...] = dis


def _tc_mid_body(p_ref, hs_ref, dis_ref, b_ref, w_ref, hs2_ref):
    dis_c = dis_ref[...][:, None]
    agg = dis_c * (p_ref[0] + p_ref[1] + hs_ref[...]) + b_ref[...][None, :]
    z = jnp.maximum(agg, 0.0)
    h2 = jnp.dot(z, w_ref[...], preferred_element_type=jnp.float32)
    hs2_ref[...] = h2 * dis_c


def _tc_out_body(p_ref, hs_ref, dis_ref, b_ref, out_ref):
    dis_c = dis_ref[...][:, None]
    out_ref[...] = dis_c * (p_ref[0] + p_ref[1] + hs_ref[...]) + b_ref[...][None, :]


def _row_spec():
    return pl.BlockSpec((BR, D), lambda i: (i, 0))


def _vec_spec():
    return pl.BlockSpec((BR,), lambda i: (i,))


def _full_spec(shape):
    nd = len(shape)
    return pl.BlockSpec(shape, lambda i: (0,) * nd)


def _tc_dense1(xp, W1, hist):
    return pl.pallas_call(
        _tc_dense1_body,
        grid=(_TC_GRID,),
        in_specs=[_row_spec(), _full_spec((D, D)), _vec_spec(), _vec_spec()],
        out_specs=[_row_spec(), _vec_spec()],
        out_shape=[
            jax.ShapeDtypeStruct((NPAD, D), jnp.float32),
            jax.ShapeDtypeStruct((NPAD,), jnp.float32),
        ],
    )(xp, W1, hist[0], hist[1])


def _tc_mid(parts, hs, dis, b1, W2):
    return pl.pallas_call(
        _tc_mid_body,
        grid=(_TC_GRID,),
        in_specs=[
            pl.BlockSpec((NC, BR, D), lambda i: (0, i, 0)),
            _row_spec(), _vec_spec(), _full_spec((D,)), _full_spec((D, D)),
        ],
        out_specs=_row_spec(),
        out_shape=jax.ShapeDtypeStruct((NPAD, D), jnp.float32),
    )(parts, hs, dis, b1, W2)


def _tc_out(parts, hs2, dis, b2):
    return pl.pallas_call(
        _tc_out_body,
        grid=(_TC_GRID,),
        in_specs=[
            pl.BlockSpec((NC, BR, D), lambda i: (0, i, 0)),
            _row_spec(), _vec_spec(), _full_spec((D,)),
        ],
        out_specs=_row_spec(),
        out_shape=jax.ShapeDtypeStruct((NPAD, D), jnp.float32),
    )(parts, hs2, dis, b2)


def kernel(x, edge_index, W1, b1, W2, b2):
    E = edge_index.shape[1]
    src = edge_index[0].astype(jnp.int32)
    dst = edge_index[1].astype(jnp.int32)
    pad = jnp.full((EPAD - E,), N, jnp.int32)
    srcp = jnp.concatenate([src, pad]).reshape(TOTB, B)
    dstp = jnp.concatenate([dst, pad]).reshape(TOTB, B)
    xp = jnp.pad(x, ((0, NPAD - N), (0, 0)))

    hist = _sc_hist(dstp)
    hs1, dis = _tc_dense1(xp, W1, hist)
    parts1 = _sc_agg(hs1, srcp, dstp)
    hs2 = _tc_mid(parts1, hs1, dis, b1, W2)
    parts2 = _sc_agg(hs2, srcp, dstp)
    outp = _tc_out(parts2, hs2, dis, b2)
    return outp[:N]


# speedup vs baseline: 3.7145x; 3.7145x over previous
import functools

import jax
import jax.numpy as jnp
from jax import lax
from jax.experimental import pallas as pl
from jax.experimental.pallas import tpu as pltpu
from jax.experimental.pallas import tpu_sc as plsc

N = 10000
NPAD = 10240
D = 128
NC = 2
NS = 16
NW = NC * NS
B = 128
TOTB = 2560
EPAD = TOTB * B
RPS = NPAD // NS
CH = 8
NCH_TOT = TOTB // CH
ZROWS = 32
Q0 = 10
Q1 = NCH_TOT // NS - Q0

_sc_mesh = plsc.VectorSubcoreMesh(core_axis_name="c", subcore_axis_name="s")


HIST_B = TOTB // NW


@functools.partial(
    pl.kernel,
    out_type=jax.ShapeDtypeStruct((NC, NPAD), jnp.float32),
    mesh=_sc_mesh,
    scratch_types=[
        pltpu.VMEM((HIST_B, B), jnp.int32),
        pltpu.VMEM((B,), jnp.float32),
        pltpu.VMEM((RPS,), jnp.float32),
        pltpu.VMEM_SHARED((NPAD,), jnp.float32),
    ],
)
def _sc_hist(dst_hbm, out_hbm, dst_v, ones_v, zb_v, hist_sh):
    c = lax.axis_index("c")
    s = lax.axis_index("s")
    wid = s * NC + c

    def fill_ones(i, carry):
        ones_v[pl.ds(i * 16, 16)] = jnp.full((16,), 1.0, jnp.float32)
        return carry

    lax.fori_loop(0, B // 16, fill_ones, 0)

    def fill_zeros(i, carry):
        zb_v[pl.ds(i * 16, 16)] = jnp.zeros((16,), jnp.float32)
        return carry

    lax.fori_loop(0, RPS // 16, fill_zeros, 0)

    pltpu.sync_copy(zb_v, hist_sh.at[pl.ds(s * RPS, RPS)])
    plsc.subcore_barrier()

    pltpu.sync_copy(dst_hbm.at[pl.ds(wid * HIST_B, HIST_B)], dst_v)

    def body(j, carry):
        pltpu.sync_copy(ones_v, hist_sh.at[dst_v.at[j]], add=True)
        return carry

    lax.fori_loop(0, HIST_B, body, 0)
    plsc.subcore_barrier()
    pltpu.sync_copy(hist_sh.at[pl.ds(s * RPS, RPS)],
                    out_hbm.at[c, pl.ds(s * RPS, RPS)])


@functools.partial(
    pl.kernel,
    out_type=jax.ShapeDtypeStruct((NC, NPAD, D), jnp.float32),
    mesh=_sc_mesh,
    scratch_types=[
        pltpu.VMEM((2, CH, B), jnp.int32),
        pltpu.VMEM((2, CH, B), jnp.int32),
        pltpu.VMEM((2, B, D), jnp.float32),
        pltpu.VMEM((ZROWS, D), jnp.float32),
        pltpu.VMEM_SHARED((NPAD, D), jnp.float32),
        pltpu.SemaphoreType.DMA,
        pltpu.SemaphoreType.DMA,
        pltpu.SemaphoreType.DMA,
        pltpu.SemaphoreType.DMA,
    ],
)
def _sc_agg(hs_hbm, src_hbm, dst_hbm, out_hbm,
            src_v, dst_v, gbuf, zb, acc_sh, sem_g0, sem_g1, sem_i0, sem_i1):
    c = lax.axis_index("c")
    s = lax.axis_index("s")
    sem_g = (sem_g0, sem_g1)
    sem_i = (sem_i0, sem_i1)
    nch = jnp.where(c == 0, Q0, Q1)
    cbase = jnp.where(c == 0, s * Q0, NS * Q0 + s * Q1)

    def issue_idx(ib, chunk):
        pltpu.async_copy(src_hbm.at[pl.ds(chunk * CH, CH)],
                         src_v.at[ib], sem_i[ib])
        pltpu.async_copy(dst_hbm.at[pl.ds(chunk * CH, CH)],
                         dst_v.at[ib], sem_i[ib])

    def wait_idx(ib):
        pltpu.make_async_copy(src_hbm.at[pl.ds(0, CH)],
                              src_v.at[ib], sem_i[ib]).wait()
        pltpu.make_async_copy(dst_hbm.at[pl.ds(0, CH)],
                              dst_v.at[ib], sem_i[ib]).wait()

    def issue_gather(k):
        pltpu.async_copy(hs_hbm.at[src_v.at[k // CH, k % CH]],
                         gbuf.at[k % 2], sem_g[k % 2])

    def wait_gather(k):
        pltpu.make_async_copy(hs_hbm.at[src_v.at[k // CH, k % CH]],
                              gbuf.at[k % 2], sem_g[k % 2]).wait()

    @pl.when(nch > 0)
    def _():
        issue_idx(0, cbase)
        issue_idx(1, cbase + 1)

    def fill_zeros(r, carry):
        for k in range(D // 16):
            zb[r, pl.ds(k * 16, 16)] = jnp.zeros((16,), jnp.float32)
        return carry

    lax.fori_loop(0, ZROWS, fill_zeros, 0)

    def zero_acc(t, carry):
        pltpu.sync_copy(zb, acc_sh.at[pl.ds(s * RPS + t * ZROWS, ZROWS)])
        return carry

    lax.fori_loop(0, RPS // ZROWS, zero_acc, 0)
    plsc.subcore_barrier()

    def body(u, carry):
        wait_idx(0)
        issue_gather(0)
        for k in range(2 * CH):
            if k + 1 < 2 * CH:
                if k + 1 == CH:
                    wait_idx(1)
                issue_gather(k + 1)
            wait_gather(k)
            pltpu.sync_copy(gbuf.at[k % 2],
                            acc_sh.at[dst_v.at[k // CH, k % CH]], add=True)
            if k == CH - 1:
                @pl.when(u < nch // 2 - 1)
                def _():
                    issue_idx(0, cbase + 2 * u + 2)

        @pl.when(u < nch // 2 - 1)
        def _():
            issue_idx(1, cbase + 2 * u + 3)
        return carry

    lax.fori_loop(0, nch // 2, body, 0)

    plsc.subcore_barrier()
    pltpu.sync_copy(acc_sh.at[pl.ds(s * RPS, RPS)],
                    out_hbm.at[c, pl.ds(s * RPS, RPS)])


BR = 1024
_TC_GRID = NPAD // BR


def _tc_dense1_body(x_ref, w_ref, h0_ref, h1_ref, hs_ref, dis_ref):
    deg = h0_ref[...] + h1_ref[...] + 1.0
    dis = lax.rsqrt(deg)
    rows = pl.program_id(0) * BR + lax.broadcasted_iota(jnp.int32, (BR,), 0)
    dis = jnp.where(rows < N, dis, 0.0)
    h = jnp.dot(x_ref[...], w_ref[...], preferred_element_type=jnp.float32)
    hs_ref[...] = h * dis[:, None]
    dis_ref[...] = dis


def _tc_mid_body(p_ref, hs_ref, dis_ref, b_ref, w_ref, hs2_ref):
    dis_c = dis_ref[...][:, None]
    agg = dis_c * (p_ref[0] + p_ref[1] + hs_ref[...]) + b_ref[...][None, :]
    z = jnp.maximum(agg, 0.0)
    h2 = jnp.dot(z, w_ref[...], preferred_element_type=jnp.float32)
    hs2_ref[...] = h2 * dis_c


def _tc_out_body(p_ref, hs_ref, dis_ref, b_ref, out_ref):
    dis_c = dis_ref[...][:, None]
    out_ref[...] = dis_c * (p_ref[0] + p_ref[1] + hs_ref[...]) + b_ref[...][None, :]


def _row_spec():
    return pl.BlockSpec((BR, D), lambda i: (i, 0))


def _vec_spec():
    return pl.BlockSpec((BR,), lambda i: (i,))


def _full_spec(shape):
    nd = len(shape)
    return pl.BlockSpec(shape, lambda i: (0,) * nd)


def _tc_dense1(xp, W1, hist):
    return pl.pallas_call(
        _tc_dense1_body,
        grid=(_TC_GRID,),
        in_specs=[_row_spec(), _full_spec((D, D)), _vec_spec(), _vec_spec()],
        out_specs=[_row_spec(), _vec_spec()],
        out_shape=[
            jax.ShapeDtypeStruct((NPAD, D), jnp.float32),
            jax.ShapeDtypeStruct((NPAD,), jnp.float32),
        ],
    )(xp, W1, hist[0], hist[1])


def _tc_mid(parts, hs, dis, b1, W2):
    return pl.pallas_call(
        _tc_mid_body,
        grid=(_TC_GRID,),
        in_specs=[
            pl.BlockSpec((NC, BR, D), lambda i: (0, i, 0)),
            _row_spec(), _vec_spec(), _full_spec((D,)), _full_spec((D, D)),
        ],
        out_specs=_row_spec(),
        out_shape=jax.ShapeDtypeStruct((NPAD, D), jnp.float32),
    )(parts, hs, dis, b1, W2)


def _tc_out(parts, hs2, dis, b2):
    return pl.pallas_call(
        _tc_out_body,
        grid=(_TC_GRID,),
        in_specs=[
            pl.BlockSpec((NC, BR, D), lambda i: (0, i, 0)),
            _row_spec(), _vec_spec(), _full_spec((D,)),
        ],
        out_specs=_row_spec(),
        out_shape=jax.ShapeDtypeStruct((NPAD, D), jnp.float32),
    )(parts, hs2, dis, b2)


def kernel(x, edge_index, W1, b1, W2, b2):
    E = edge_index.shape[1]
    src = edge_index[0].astype(jnp.int32)
    dst = edge_index[1].astype(jnp.int32)
    pad = N + jnp.arange(EPAD - E, dtype=jnp.int32) % (NPAD - N)
    srcp = jnp.concatenate([src, pad]).reshape(TOTB, B)
    dstp = jnp.concatenate([dst, pad]).reshape(TOTB, B)
    xp = jnp.pad(x, ((0, NPAD - N), (0, 0)))

    hist = _sc_hist(dstp)
    hs1, dis = _tc_dense1(xp, W1, hist)
    parts1 = _sc_agg(hs1, srcp, dstp)
    hs2 = _tc_mid(parts1, hs1, dis, b1, W2)
    parts2 = _sc_agg(hs2, srcp, dstp)
    outp = _tc_out(parts2, hs2, dis, b2)
    return outp[:N]


# speedup vs baseline: 3.7265x; 1.0032x over previous
import functools

import jax
import jax.numpy as jnp
import numpy as np
from jax import lax
from jax.experimental import pallas as pl
from jax.experimental.pallas import tpu as pltpu
from jax.experimental.pallas import tpu_sc as plsc

N = 10000
NPAD = 10240
D = 128
NC = 2
NS = 16
NW = NC * NS
B = 128
TOTB = 2560
EPAD = TOTB * B
RPS = NPAD // NS
CH = 8
NCH_TOT = TOTB // CH
ZROWS = 32
Q0 = 10
Q1 = NCH_TOT // NS - Q0

_sc_mesh = plsc.VectorSubcoreMesh(core_axis_name="c", subcore_axis_name="s")


HIST_B = TOTB // NW


@functools.partial(
    pl.kernel,
    out_type=jax.ShapeDtypeStruct((NC, NPAD), jnp.float32),
    mesh=_sc_mesh,
    scratch_types=[
        pltpu.VMEM((HIST_B, B), jnp.int32),
        pltpu.VMEM((B,), jnp.float32),
        pltpu.VMEM((RPS,), jnp.float32),
        pltpu.VMEM_SHARED((NPAD,), jnp.float32),
    ],
)
def _sc_hist(dst_hbm, out_hbm, dst_v, ones_v, zb_v, hist_sh):
    c = lax.axis_index("c")
    s = lax.axis_index("s")
    wid = s * NC + c

    def fill_ones(i, carry):
        ones_v[pl.ds(i * 16, 16)] = jnp.full((16,), 1.0, jnp.float32)
        return carry

    lax.fori_loop(0, B // 16, fill_ones, 0)

    def fill_zeros(i, carry):
        zb_v[pl.ds(i * 16, 16)] = jnp.zeros((16,), jnp.float32)
        return carry

    lax.fori_loop(0, RPS // 16, fill_zeros, 0)

    pltpu.sync_copy(zb_v, hist_sh.at[pl.ds(s * RPS, RPS)])
    plsc.subcore_barrier()

    pltpu.sync_copy(dst_hbm.at[pl.ds(wid * HIST_B, HIST_B)], dst_v)

    def body(j, carry):
        pltpu.sync_copy(ones_v, hist_sh.at[dst_v.at[j]], add=True)
        return carry

    lax.fori_loop(0, HIST_B, body, 0)
    plsc.subcore_barrier()
    pltpu.sync_copy(hist_sh.at[pl.ds(s * RPS, RPS)],
                    out_hbm.at[c, pl.ds(s * RPS, RPS)])


@functools.partial(
    pl.kernel,
    out_type=jax.ShapeDtypeStruct((NC, NPAD, D), jnp.float32),
    mesh=_sc_mesh,
    scratch_types=[
        pltpu.VMEM((2, CH, B), jnp.int32),
        pltpu.VMEM((2, CH, B), jnp.int32),
        pltpu.VMEM((2, B, D), jnp.float32),
        pltpu.VMEM((ZROWS, D), jnp.float32),
        pltpu.VMEM_SHARED((NPAD, D), jnp.float32),
        pltpu.SemaphoreType.DMA,
        pltpu.SemaphoreType.DMA,
        pltpu.SemaphoreType.DMA,
        pltpu.SemaphoreType.DMA,
        pltpu.SemaphoreType.DMA,
        pltpu.SemaphoreType.DMA,
    ],
)
def _sc_agg(hs_hbm, src_hbm, dst_hbm, out_hbm,
            src_v, dst_v, gbuf, zb, acc_sh,
            sem_g0, sem_g1, sem_i0, sem_i1, sem_s0, sem_s1):
    c = lax.axis_index("c")
    s = lax.axis_index("s")
    sem_g = (sem_g0, sem_g1)
    sem_i = (sem_i0, sem_i1)
    sem_s = (sem_s0, sem_s1)
    nch = jnp.where(c == 0, Q0, Q1)
    cbase = jnp.where(c == 0, s * Q0, NS * Q0 + s * Q1)

    def issue_idx(ib, chunk):
        pltpu.async_copy(src_hbm.at[pl.ds(chunk * CH, CH)],
                         src_v.at[ib], sem_i[ib])
        pltpu.async_copy(dst_hbm.at[pl.ds(chunk * CH, CH)],
                         dst_v.at[ib], sem_i[ib])

    def wait_idx(ib):
        pltpu.make_async_copy(src_hbm.at[pl.ds(0, CH)],
                              src_v.at[ib], sem_i[ib]).wait()
        pltpu.make_async_copy(dst_hbm.at[pl.ds(0, CH)],
                              dst_v.at[ib], sem_i[ib]).wait()

    def issue_gather(k):
        pltpu.async_copy(hs_hbm.at[src_v.at[k // CH, k % CH]],
                         gbuf.at[k % 2], sem_g[k % 2])

    def wait_gather(k):
        pltpu.make_async_copy(hs_hbm.at[src_v.at[k // CH, k % CH]],
                              gbuf.at[k % 2], sem_g[k % 2]).wait()

    def issue_scatter(k):
        pltpu.async_copy(gbuf.at[k % 2],
                         acc_sh.at[dst_v.at[k // CH, k % CH]],
                         sem_s[k % 2], add=True)

    def wait_scatter(k):
        pltpu.make_async_copy(gbuf.at[k % 2],
                              acc_sh.at[dst_v.at[k // CH, k % CH]],
                              sem_s[k % 2]).wait()

    @pl.when(nch > 0)
    def _():
        issue_idx(0, cbase)
        issue_idx(1, cbase + 1)

    def fill_zeros(r, carry):
        for k in range(D // 16):
            zb[r, pl.ds(k * 16, 16)] = jnp.zeros((16,), jnp.float32)
        return carry

    lax.fori_loop(0, ZROWS, fill_zeros, 0)

    def zero_acc(t, carry):
        pltpu.sync_copy(zb, acc_sh.at[pl.ds(s * RPS + t * ZROWS, ZROWS)])
        return carry

    lax.fori_loop(0, RPS // ZROWS, zero_acc, 0)
    plsc.subcore_barrier()

    def body(u, carry):
        wait_idx(0)
        issue_gather(0)
        for k in range(2 * CH):
            if k + 1 < 2 * CH:
                if k + 1 == CH:
                    wait_idx(1)
                if k >= 1:
                    wait_scatter(k - 1)
                issue_gather(k + 1)
            wait_gather(k)
            issue_scatter(k)
            if k == CH:
                @pl.when(u < nch // 2 - 1)
                def _():
                    issue_idx(0, cbase + 2 * u + 2)

        wait_scatter(2 * CH - 2)
        wait_scatter(2 * CH - 1)

        @pl.when(u < nch // 2 - 1)
        def _():
            issue_idx(1, cbase + 2 * u + 3)
        return carry

    lax.fori_loop(0, nch // 2, body, 0)

    plsc.subcore_barrier()
    pltpu.sync_copy(acc_sh.at[pl.ds(s * RPS, RPS)],
                    out_hbm.at[c, pl.ds(s * RPS, RPS)])


BR = 1024
_TC_GRID = NPAD // BR


def _tc_dense1_body(x_ref, w_ref, h0_ref, h1_ref, hs_ref, dis_ref):
    deg = h0_ref[...] + h1_ref[...] + 1.0
    dis = lax.rsqrt(deg)
    rows = pl.program_id(0) * BR + lax.broadcasted_iota(jnp.int32, (BR,), 0)
    dis = jnp.where(rows < N, dis, 0.0)
    h = jnp.dot(x_ref[...], w_ref[...], preferred_element_type=jnp.float32)
    hs_ref[...] = h * dis[:, None]
    dis_ref[...] = dis


def _tc_mid_body(p_ref, hs_ref, dis_ref, b_ref, w_ref, hs2_ref):
    dis_c = dis_ref[...][:, None]
    agg = dis_c * (p_ref[0] + p_ref[1] + hs_ref[...]) + b_ref[...][None, :]
    z = jnp.maximum(agg, 0.0)
    h2 = jnp.dot(z, w_ref[...], preferred_element_type=jnp.float32)
    hs2_ref[...] = h2 * dis_c


def _tc_out_body(p_ref, hs_ref, dis_ref, b_ref, out_ref):
    dis_c = dis_ref[...][:, None]
    out_ref[...] = dis_c * (p_ref[0] + p_ref[1] + hs_ref[...]) + b_ref[...][None, :]


def _row_spec():
    return pl.BlockSpec((BR, D), lambda i: (i, 0))


def _vec_spec():
    return pl.BlockSpec((BR,), lambda i: (i,))


def _full_spec(shape):
    nd = len(shape)
    return pl.BlockSpec(shape, lambda i: (0,) * nd)


def _tc_dense1(xp, W1, hist):
    return pl.pallas_call(
        _tc_dense1_body,
        grid=(_TC_GRID,),
        in_specs=[_row_spec(), _full_spec((D, D)), _vec_spec(), _vec_spec()],
        out_specs=[_row_spec(), _vec_spec()],
        out_shape=[
            jax.ShapeDtypeStruct((NPAD, D), jnp.float32),
            jax.ShapeDtypeStruct((NPAD,), jnp.float32),
        ],
    )(xp, W1, hist[0], hist[1])


def _tc_mid(parts, hs, dis, b1, W2):
    return pl.pallas_call(
        _tc_mid_body,
        grid=(_TC_GRID,),
        in_specs=[
            pl.BlockSpec((NC, BR, D), lambda i: (0, i, 0)),
            _row_spec(), _vec_spec(), _full_spec((D,)), _full_spec((D, D)),
        ],
        out_specs=_row_spec(),
        out_shape=jax.ShapeDtypeStruct((NPAD, D), jnp.float32),
    )(parts, hs, dis, b1, W2)


def _tc_out(parts, hs2, dis, b2):
    return pl.pallas_call(
        _tc_out_body,
        grid=(_TC_GRID,),
        in_specs=[
            pl.BlockSpec((NC, BR, D), lambda i: (0, i, 0)),
            _row_spec(), _vec_spec(), _full_spec((D,)),
        ],
        out_specs=_row_spec(),
        out_shape=jax.ShapeDtypeStruct((NPAD, D), jnp.float32),
    )(parts, hs2, dis, b2)


def kernel(x, edge_index, W1, b1, W2, b2):
    E = edge_index.shape[1]
    src = edge_index[0].astype(jnp.int32)
    dst = edge_index[1].astype(jnp.int32)
    pad = jnp.asarray(N + np.arange(EPAD - E, dtype=np.int32) % (NPAD - N))
    srcp = jnp.concatenate([src, pad]).reshape(TOTB, B)
    dstp = jnp.concatenate([dst, pad]).reshape(TOTB, B)
    xp = jnp.pad(x, ((0, NPAD - N), (0, 0)))

    hist = _sc_hist(dstp)
    hs1, dis = _tc_dense1(xp, W1, hist)
    parts1 = _sc_agg(hs1, srcp, dstp)
    hs2 = _tc_mid(parts1, hs1, dis, b1, W2)
    parts2 = _sc_agg(hs2, srcp, dstp)
    return _tc_out(parts2, hs2, dis, b2)[:N]
